# baseline (speedup 1.0000x reference)
import functools
import math

import jax
import jax.numpy as jnp
from jax import lax
from jax.experimental import pallas as pl
from jax.experimental.pallas import tpu as pltpu

_INV_SQRT2 = 1.0 / math.sqrt(2.0)


def _gelu_exact(x):
    return 0.5 * x * (1.0 + lax.erf(x * _INV_SQRT2))


def _ffn_kernel(x_hbm, w1_hbm, b1_ref, w2_hbm, b2_ref, o_hbm,
                xin, xb, w1l, w2l, w1b, w2b, acc,
                sx, sw1, sw2, so, *, nk, nj, th, tmj, rows_core):
    i = pl.program_id(0)
    r0 = i * rows_core

    def x_copy(j, slot):
        return pltpu.make_async_copy(
            x_hbm.at[pl.ds(r0 + j * tmj, tmj), :], xin.at[slot], sx.at[slot])

    def w1_copy(k, slot):
        return pltpu.make_async_copy(
            w1_hbm.at[:, pl.ds(k * th, th)], w1l.at[slot], sw1.at[slot])

    def w2_copy(k, slot):
        return pltpu.make_async_copy(
            w2_hbm.at[pl.ds(k * th, th), :], w2l.at[slot], sw2.at[slot])

    def o_copy(j):
        return pltpu.make_async_copy(
            acc.at[pl.ds(j * tmj, tmj), :],
            o_hbm.at[pl.ds(r0 + j * tmj, tmj), :], so.at[j])

    x_copy(0, 0).start()
    w1_copy(0, 0).start()
    w2_copy(0, 0).start()

    x_copy(0, 0).wait()
    if nj > 1:
        x_copy(1, 1).start()
    xb[pl.ds(0, tmj), :] = xin[0].astype(jnp.bfloat16)

    b2f = b2_ref[...].astype(jnp.float32)

    def subtile(j):
        rows = pl.ds(j * tmj, tmj)
        o = jnp.broadcast_to(b2f, (tmj, b2f.shape[1]))
        for k in range(nk):
            if j == 0:
                w1_copy(k, k % 2).wait()
                w2_copy(k, k % 2).wait()
                w1b[k] = w1l[k % 2].astype(jnp.bfloat16)
                w2b[k] = w2l[k % 2].astype(jnp.bfloat16)
                if k + 1 < nk:
                    w1_copy(k + 1, (k + 1) % 2).start()
                    w2_copy(k + 1, (k + 1) % 2).start()
            h = jnp.dot(xb[rows, :], w1b[k],
                        preferred_element_type=jnp.float32)
            h = _gelu_exact(h + b1_ref[:, pl.ds(k * th, th)].astype(jnp.float32))
            o = jnp.dot(h.astype(jnp.bfloat16), w2b[k],
                        preferred_element_type=jnp.float32) + o
        acc[rows, :] = o
        o_copy(j).start()

    subtile(0)
    for j in range(1, nj):
        x_copy(j, j % 2).wait()
        if j + 1 < nj:
            x_copy(j + 1, (j + 1) % 2).start()
        xb[pl.ds(j * tmj, tmj), :] = xin[j % 2].astype(jnp.bfloat16)
        subtile(j)

    for j in range(nj):
        o_copy(j).wait()


def kernel(x, w1, b1, w2, b2):
    batch, seq, dim = x.shape
    hidden = w1.shape[1]
    M = batch * seq
    x2d = x.reshape(M, dim)

    b1r = b1.reshape(1, hidden).astype(jnp.float32)
    b2r = b2.reshape(1, dim).astype(jnp.float32)

    nj = 4
    tmj = 512
    Mp = -(-M // (2 * nj * tmj)) * (2 * nj * tmj)
    if Mp != M:
        x2d = jnp.pad(x2d, ((0, Mp - M), (0, 0)))
    rows_core = Mp // 2
    tmj = rows_core // nj

    th = 1024 if hidden % 1024 == 0 else hidden
    nk = hidden // th

    cost = pl.CostEstimate(
        flops=int(4 * Mp * dim * hidden),
        transcendentals=int(Mp * hidden),
        bytes_accessed=int(4 * Mp * dim * 2 + 2 * (dim * hidden * 4)),
    )

    out2d = pl.pallas_call(
        functools.partial(_ffn_kernel, nk=nk, nj=nj, th=th, tmj=tmj,
                          rows_core=rows_core),
        out_shape=jax.ShapeDtypeStruct((Mp, dim), x.dtype),
        grid=(2,),
        in_specs=[
            pl.BlockSpec(memory_space=pl.ANY),
            pl.BlockSpec(memory_space=pl.ANY),
            pl.BlockSpec((1, hidden), lambda i: (0, 0)),
            pl.BlockSpec(memory_space=pl.ANY),
            pl.BlockSpec((1, dim), lambda i: (0, 0)),
        ],
        out_specs=pl.BlockSpec(memory_space=pl.ANY),
        scratch_shapes=[
            pltpu.VMEM((2, tmj, dim), jnp.float32),
            pltpu.VMEM((rows_core, dim), jnp.bfloat16),
            pltpu.VMEM((2, dim, th), jnp.float32),
            pltpu.VMEM((2, th, dim), jnp.float32),
            pltpu.VMEM((nk, dim, th), jnp.bfloat16),
            pltpu.VMEM((nk, th, dim), jnp.bfloat16),
            pltpu.VMEM((rows_core, dim), jnp.float32),
            pltpu.SemaphoreType.DMA((2,)),
            pltpu.SemaphoreType.DMA((2,)),
            pltpu.SemaphoreType.DMA((2,)),
            pltpu.SemaphoreType.DMA((4,)),
        ],
        compiler_params=pltpu.CompilerParams(
            dimension_semantics=("parallel",),
            vmem_limit_bytes=int(64 * 1024 * 1024 * 0.9),
        ),
        cost_estimate=cost,
    )(x2d, w1, b1r, w2, b2r)

    if Mp != M:
        out2d = out2d[:M]
    return out2d.reshape(batch, seq, dim)
